# Initial kernel scaffold; baseline (speedup 1.0000x reference)
import jax
import jax.numpy as jnp
from jax.experimental import pallas as pl


def kernel(x, edge_index, edge_weight, W1, b1, gamma1, beta1, W2, b2, Wc, bc):
    raise NotImplementedError("write your pallas kernel here")


# speedup vs baseline: 10.8790x; 10.8790x over previous
import functools

import jax
import jax.numpy as jnp
from jax import lax
from jax.experimental import pallas as pl
from jax.experimental.pallas import tpu as pltpu
from jax.experimental.pallas import tpu_sc as plsc

N = 10000
E = 320000
D = 128
H = 128
OUT = 70

NC = 2
NS = 16
NW = NC * NS
EPT = E // NW

KD = 100
DCH = EPT // KD
NPAD = 10240

K = 16
EPTP = 10240
ROWS = EPTP // 128
NCH = EPTP // K
NBUF = 5
SHIFT = 14
SMASK = (1 << SHIFT) - 1

_mesh = plsc.VectorSubcoreMesh(core_axis_name="c", subcore_axis_name="s")


@functools.partial(
    pl.kernel,
    out_type=jax.ShapeDtypeStruct((NC, NPAD), jnp.float32),
    mesh=_mesh,
    scratch_types=[
        pltpu.VMEM((DCH, KD), jnp.int32),
        pltpu.VMEM((DCH, KD), jnp.float32),
        pltpu.VMEM((640,), jnp.float32),
        pltpu.VMEM_SHARED((NPAD,), jnp.float32),
        pltpu.SemaphoreType.DMA,
    ],
)
def _sc_degree(dst_hbm, ew_hbm, deg_out, dst_v, ew_v, zbuf, acc, sem):
    c = lax.axis_index("c")
    s = lax.axis_index("s")
    w = s * NC + c

    pltpu.sync_copy(dst_hbm.at[w], dst_v)
    pltpu.sync_copy(ew_hbm.at[w], ew_v)

    @pl.loop(0, 40)
    def _z(i):
        zbuf[pl.ds(i * 16, 16)] = jnp.zeros((16,), jnp.float32)

    pltpu.sync_copy(zbuf, acc.at[pl.ds(s * 640, 640)])
    plsc.subcore_barrier()

    @pl.loop(0, DCH)
    def _fire(j):
        pltpu.async_copy(ew_v.at[j], acc.at[dst_v.at[j]], sem, add=True)

    @pl.loop(0, DCH)
    def _drain(j):
        pltpu.make_async_copy(ew_v.at[0], acc.at[dst_v.at[0]], sem).wait()

    plsc.subcore_barrier()

    pltpu.sync_copy(acc.at[pl.ds(s * 640, 640)],
                    deg_out.at[c].at[pl.ds(s * 640, 640)])


@functools.partial(
    pl.kernel,
    out_type=jax.ShapeDtypeStruct((NC, N, H), jnp.float32),
    mesh=_mesh,
    scratch_types=[
        pltpu.VMEM((ROWS, 128), jnp.int32),
        pltpu.VMEM((ROWS, 128), jnp.float32),
        [pltpu.VMEM((K, H), jnp.float32) for _ in range(NBUF)],
        [pltpu.VMEM((K, H), jnp.float32) for _ in range(NBUF)],
        pltpu.SemaphoreType.DMA((NBUF,)),
        pltpu.SemaphoreType.DMA((NBUF,)),
        pltpu.SemaphoreType.DMA,
        pltpu.VMEM_SHARED((N, H), jnp.float32),
    ],
)
def _sc_aggregate(hp_hbm, pk_hbm, ew_hbm, s_out,
                  pk_v, ew_v, gbufs, sbufs, gsem, ssem, zsem, acc):
    c = lax.axis_index("c")
    s = lax.axis_index("s")
    w = s * NC + c

    pltpu.sync_copy(pk_hbm.at[w], pk_v)
    pltpu.sync_copy(ew_hbm.at[w], ew_v)

    for r in range(K):
        for q in range(8):
            sbufs[0][r, pl.ds(q * 16, 16)] = jnp.zeros((16,), jnp.float32)

    @pl.when(s < 10)
    def _zero():
        @pl.loop(0, 62)
        def _zf(kk):
            pltpu.async_copy(sbufs[0], acc.at[pl.ds(s * 1000 + kk * 16, 16)],
                             zsem)
        pltpu.async_copy(sbufs[0].at[pl.ds(0, 8)],
                         acc.at[pl.ds(s * 1000 + 992, 8)], zsem)

        @pl.loop(0, 62)
        def _zd(kk):
            pltpu.make_async_copy(sbufs[0], acc.at[pl.ds(0, 16)], zsem).wait()
        pltpu.make_async_copy(sbufs[0].at[pl.ds(0, 8)], acc.at[pl.ds(0, 8)],
                              zsem).wait()

    plsc.subcore_barrier()

    def _chunk(ref, j):
        row = jnp.right_shift(j, 3)
        lane = jnp.bitwise_and(j, 7) * 16
        return ref[row, pl.ds(lane, 16)]

    for t in range(NBUF):
        sv0 = jnp.bitwise_and(_chunk(pk_v, t), SMASK)
        pltpu.async_copy(hp_hbm.at[sv0], gbufs[t], gsem.at[t])

    @pl.loop(0, NCH, step=NBUF)
    def _outer(o):
        for t in range(NBUF):
            j = o + t
            sv = jnp.bitwise_and(_chunk(pk_v, j), SMASK)
            pltpu.make_async_copy(hp_hbm.at[sv], gbufs[t], gsem.at[t]).wait()

            @pl.when(j >= NBUF)
            def _w():
                dvp = jnp.right_shift(_chunk(pk_v, j - NBUF), SHIFT)
                pltpu.make_async_copy(sbufs[t], acc.at[dvp],
                                      ssem.at[t]).wait()

            cvec = _chunk(ew_v, j)
            for r in range(K):
                cval = cvec[r]
                for q in range(8):
                    sbufs[t][r, pl.ds(q * 16, 16)] = (
                        gbufs[t][r, pl.ds(q * 16, 16)] * cval)

            dv = jnp.right_shift(_chunk(pk_v, j), SHIFT)
            pltpu.async_copy(sbufs[t], acc.at[dv], ssem.at[t], add=True)

            @pl.when(j + NBUF < NCH)
            def _g():
                sv2 = jnp.bitwise_and(_chunk(pk_v, j + NBUF), SMASK)
                pltpu.async_copy(hp_hbm.at[sv2], gbufs[t], gsem.at[t])

    for t in range(NBUF):
        dvl = jnp.right_shift(_chunk(pk_v, NCH - NBUF + t), SHIFT)
        pltpu.make_async_copy(sbufs[t], acc.at[dvl], ssem.at[t]).wait()

    plsc.subcore_barrier()

    @pl.when(s < 10)
    def _wb():
        pltpu.sync_copy(acc.at[pl.ds(s * 1000, 1000)],
                        s_out.at[c].at[pl.ds(s * 1000, 1000)])


_BLK = 1000
_GRID = N // _BLK


def _tc1_body(deg_ref, x_ref, w1_ref, dinv_ref, h1_ref, hp1_ref):
    dsum = deg_ref[:, 0:1] + deg_ref[:, 1:2] + 1.0
    dinv = lax.rsqrt(dsum)
    h1 = jnp.dot(x_ref[...], w1_ref[...], preferred_element_type=jnp.float32)
    dinv_ref[...] = dinv
    h1_ref[...] = h1
    hp1_ref[...] = h1 * dinv


def _tc2_body(s_ref, h1_ref, dinv_ref, b1_ref, g1_ref, be1_ref, w2_ref,
              h2_ref, hp2_ref):
    dinv = dinv_ref[...]
    agg = dinv * (s_ref[0] + s_ref[1]) + (dinv * dinv) * h1_ref[...] \
        + b1_ref[...]
    z = jnp.maximum(agg * g1_ref[...] + be1_ref[...], 0.0)
    h2 = jnp.dot(z, w2_ref[...], preferred_element_type=jnp.float32)
    h2_ref[...] = h2
    hp2_ref[...] = h2 * dinv


def _tc3_body(s_ref, h2_ref, dinv_ref, b2_ref, wc_ref, bc_ref, out_ref):
    dinv = dinv_ref[...]
    agg = dinv * (s_ref[0] + s_ref[1]) + (dinv * dinv) * h2_ref[...] \
        + b2_ref[...]
    out_ref[...] = jnp.dot(agg, wc_ref[...],
                           preferred_element_type=jnp.float32) + bc_ref[...]


def _row_spec(shape_minor):
    return pl.BlockSpec((_BLK,) + shape_minor, lambda i: (i,) + (0,) * len(shape_minor))


def _full_spec(shape):
    return pl.BlockSpec(shape, lambda i: (0,) * len(shape))


def _tc1(deg_t, x, w1):
    return pl.pallas_call(
        _tc1_body,
        grid=(_GRID,),
        in_specs=[_row_spec((NC,)), _row_spec((D,)), _full_spec((D, H))],
        out_specs=[_row_spec((1,)), _row_spec((H,)), _row_spec((H,))],
        out_shape=[
            jax.ShapeDtypeStruct((N, 1), jnp.float32),
            jax.ShapeDtypeStruct((N, H), jnp.float32),
            jax.ShapeDtypeStruct((N, H), jnp.float32),
        ],
    )(deg_t, x, w1)


def _tc2(s1, h1, dinv, b1, g1, be1, w2):
    sspec = pl.BlockSpec((NC, _BLK, H), lambda i: (0, i, 0))
    return pl.pallas_call(
        _tc2_body,
        grid=(_GRID,),
        in_specs=[sspec, _row_spec((H,)), _row_spec((1,)),
                  _full_spec((1, H)), _full_spec((1, H)), _full_spec((1, H)),
                  _full_spec((H, H))],
        out_specs=[_row_spec((H,)), _row_spec((H,))],
        out_shape=[
            jax.ShapeDtypeStruct((N, H), jnp.float32),
            jax.ShapeDtypeStruct((N, H), jnp.float32),
        ],
    )(s1, h1, dinv, b1, g1, be1, w2)


def _tc3(s2, h2, dinv, b2, wc, bc):
    sspec = pl.BlockSpec((NC, _BLK, H), lambda i: (0, i, 0))
    return pl.pallas_call(
        _tc3_body,
        grid=(_GRID,),
        in_specs=[sspec, _row_spec((H,)), _row_spec((1,)),
                  _full_spec((1, H)), _full_spec((H, OUT)),
                  _full_spec((1, OUT))],
        out_specs=_row_spec((OUT,)),
        out_shape=jax.ShapeDtypeStruct((N, OUT), jnp.float32),
    )(s2, h2, dinv, b2, wc, bc)


def kernel(x, edge_index, edge_weight, W1, b1, gamma1, beta1, W2, b2, Wc, bc):
    src = edge_index[0]
    dst = edge_index[1]

    dst_d = dst.reshape(NW, DCH, KD)
    ew_d = edge_weight.reshape(NW, DCH, KD)
    npad = EPTP - EPT
    pad_dst = jnp.broadcast_to((jnp.arange(npad, dtype=jnp.int32) * 41) % N,
                               (NW, npad))
    pk = jnp.bitwise_or(jnp.left_shift(dst, SHIFT), src).reshape(NW, EPT)
    pk_a = jnp.concatenate(
        [pk, jnp.left_shift(pad_dst, SHIFT)], axis=1).reshape(NW, ROWS, 128)
    ew_a = jnp.concatenate(
        [edge_weight.reshape(NW, EPT),
         jnp.zeros((NW, npad), jnp.float32)], axis=1).reshape(NW, ROWS, 128)

    deg_p = _sc_degree(dst_d, ew_d)
    deg_t = jnp.transpose(deg_p[:, :N])

    b1r = b1.reshape(1, H)
    g1r = gamma1.reshape(1, H)
    be1r = beta1.reshape(1, H)
    b2r = b2.reshape(1, H)
    bcr = bc.reshape(1, OUT)

    dinv, h1, hp1 = _tc1(deg_t, x, W1)
    s1 = _sc_aggregate(hp1, pk_a, ew_a)
    h2, hp2 = _tc2(s1, h1, dinv, b1r, g1r, be1r, W2)
    s2 = _sc_aggregate(hp2, pk_a, ew_a)
    out = _tc3(s2, h2, dinv, b2r, Wc, bcr)
    return out


# speedup vs baseline: 19.3231x; 1.7762x over previous
import functools

import jax
import jax.numpy as jnp
from jax import lax
from jax.experimental import pallas as pl
from jax.experimental.pallas import tpu as pltpu
from jax.experimental.pallas import tpu_sc as plsc

N = 10000
E = 320000
D = 128
H = 128
OUT = 70

NC = 2
NS = 16
NW = NC * NS
EPT = E // NW

KD = 100
DCH = EPT // KD
NPAD = 10240

K = 80
EPTP = 10240
NCHK = EPTP // K
NDAT = 2
NIDX = 4
NSLOT = 4

_mesh = plsc.VectorSubcoreMesh(core_axis_name="c", subcore_axis_name="s")


@functools.partial(
    pl.kernel,
    out_type=jax.ShapeDtypeStruct((NC, NPAD), jnp.float32),
    mesh=_mesh,
    scratch_types=[
        pltpu.VMEM((DCH, KD), jnp.int32),
        pltpu.VMEM((DCH, KD), jnp.float32),
        pltpu.VMEM((640,), jnp.float32),
        pltpu.VMEM_SHARED((NPAD,), jnp.float32),
        pltpu.SemaphoreType.DMA,
    ],
)
def _sc_degree(dst_hbm, ew_hbm, deg_out, dst_v, ew_v, zbuf, acc, sem):
    c = lax.axis_index("c")
    s = lax.axis_index("s")
    w = s * NC + c

    pltpu.sync_copy(dst_hbm.at[w], dst_v)
    pltpu.sync_copy(ew_hbm.at[w], ew_v)

    @pl.loop(0, 40)
    def _z(i):
        zbuf[pl.ds(i * 16, 16)] = jnp.zeros((16,), jnp.float32)

    pltpu.sync_copy(zbuf, acc.at[pl.ds(s * 640, 640)])
    plsc.subcore_barrier()

    @pl.loop(0, DCH)
    def _fire(j):
        pltpu.async_copy(ew_v.at[j], acc.at[dst_v.at[j]], sem, add=True)

    @pl.loop(0, DCH)
    def _drain(j):
        pltpu.make_async_copy(ew_v.at[0], acc.at[dst_v.at[0]], sem).wait()

    plsc.subcore_barrier()

    pltpu.sync_copy(acc.at[pl.ds(s * 640, 640)],
                    deg_out.at[c].at[pl.ds(s * 640, 640)])


@functools.partial(
    pl.kernel,
    out_type=jax.ShapeDtypeStruct((NC, N, H), jnp.float32),
    mesh=_mesh,
    scratch_types=[
        [pltpu.VMEM((K,), jnp.int32) for _ in range(NIDX)],
        [pltpu.VMEM((K,), jnp.int32) for _ in range(NIDX)],
        [pltpu.VMEM((K,), jnp.float32) for _ in range(NIDX)],
        [pltpu.VMEM((K, H), jnp.float32) for _ in range(NDAT)],
        [pltpu.VMEM((K, H), jnp.float32) for _ in range(NDAT)],
        pltpu.SemaphoreType.DMA((NIDX,)),
        pltpu.SemaphoreType.DMA((NIDX,)),
        pltpu.SemaphoreType.DMA((NDAT,)),
        pltpu.SemaphoreType.DMA((NDAT,)),
        pltpu.SemaphoreType.DMA,
        pltpu.VMEM_SHARED((N, H), jnp.float32),
    ],
)
def _sc_aggregate(hp_hbm, src_hbm, dst_hbm, ew_hbm, s_out,
                  src_b, dst_b, ew_b, gbufs, sbufs,
                  isem, dsem, gsem, ssem, zsem, acc):
    c = lax.axis_index("c")
    s = lax.axis_index("s")
    w = s * NC + c
    ebase = pl.multiple_of(w * EPTP, 8)

    def _edge_slice(ref, j):
        return ref.at[pl.ds(pl.multiple_of(ebase + j * K, 8), K)]

    for r in range(40):
        for q in range(8):
            sbufs[0][r, pl.ds(q * 16, 16)] = jnp.zeros((16,), jnp.float32)

    @pl.when(s < 10)
    def _zero():
        @pl.loop(0, 25)
        def _zf(kk):
            pltpu.async_copy(sbufs[0].at[pl.ds(0, 40)],
                             acc.at[pl.ds(s * 1000 + kk * 40, 40)], zsem)

        @pl.loop(0, 25)
        def _zd(kk):
            pltpu.make_async_copy(sbufs[0].at[pl.ds(0, 40)],
                                  acc.at[pl.ds(0, 40)], zsem).wait()

    plsc.subcore_barrier()

    for m in range(NIDX):
        pltpu.async_copy(_edge_slice(src_hbm, m), src_b[m], isem.at[m])
        pltpu.async_copy(_edge_slice(ew_hbm, m), ew_b[m], isem.at[m])
    for m in range(NDAT):
        pltpu.async_copy(_edge_slice(dst_hbm, m), dst_b[m], dsem.at[m])
    for m in range(NDAT):
        pltpu.make_async_copy(_edge_slice(src_hbm, m), src_b[m],
                              isem.at[m]).wait()
        pltpu.make_async_copy(_edge_slice(ew_hbm, m), ew_b[m],
                              isem.at[m]).wait()
        pltpu.async_copy(hp_hbm.at[src_b[m]], gbufs[m], gsem.at[m])

    @pl.loop(0, NCHK, step=NSLOT)
    def _outer(o):
        for t in range(NSLOT):
            j = o + t
            t2 = t % NDAT
            t4 = t % NIDX

            pltpu.make_async_copy(hp_hbm.at[src_b[t4]], gbufs[t2],
                                  gsem.at[t2]).wait()

            @pl.when(j >= NDAT)
            def _ws():
                pltpu.make_async_copy(sbufs[t2], acc.at[dst_b[t4]],
                                      ssem.at[t2]).wait()

            @pl.when(j + NDAT < NCHK)
            def _rd():
                m2 = (t + NDAT) % NIDX
                pltpu.async_copy(_edge_slice(dst_hbm, j + NDAT), dst_b[m2],
                                 dsem.at[m2])

            @pl.loop(0, K // 16)
            def _grp(g):
                b16 = g * 16
                cvec = ew_b[t4][pl.ds(b16, 16)]
                for r in range(16):
                    cval = cvec[r]
                    for q in range(8):
                        sbufs[t2][b16 + r, pl.ds(q * 16, 16)] = (
                            gbufs[t2][b16 + r, pl.ds(q * 16, 16)] * cval)

            pltpu.make_async_copy(_edge_slice(dst_hbm, j), dst_b[t4],
                                  dsem.at[t4]).wait()
            pltpu.async_copy(sbufs[t2], acc.at[dst_b[t4]], ssem.at[t2],
                             add=True)

            @pl.when(j + NDAT < NCHK)
            def _g2():
                m2 = (t + NDAT) % NIDX
                pltpu.make_async_copy(_edge_slice(src_hbm, j + NDAT),
                                      src_b[m2], isem.at[m2]).wait()
                pltpu.make_async_copy(_edge_slice(ew_hbm, j + NDAT),
                                      ew_b[m2], isem.at[m2]).wait()
                pltpu.async_copy(hp_hbm.at[src_b[m2]], gbufs[t2],
                                 gsem.at[t2])

            @pl.when(j + NIDX < NCHK)
            def _ri():
                pltpu.async_copy(_edge_slice(src_hbm, j + NIDX), src_b[t4],
                                 isem.at[t4])
                pltpu.async_copy(_edge_slice(ew_hbm, j + NIDX), ew_b[t4],
                                 isem.at[t4])

    for t in range(NDAT):
        t4 = (NCHK - NDAT + t) % NIDX
        pltpu.make_async_copy(sbufs[t], acc.at[dst_b[t4]],
                              ssem.at[t]).wait()

    plsc.subcore_barrier()

    @pl.when(s < 10)
    def _wb():
        pltpu.sync_copy(acc.at[pl.ds(s * 1000, 1000)],
                        s_out.at[c].at[pl.ds(s * 1000, 1000)])


_BLK = 1000
_GRID = N // _BLK


def _tc1_body(deg_ref, x_ref, w1_ref, dinv_ref, h1_ref, hp1_ref):
    dsum = deg_ref[:, 0:1] + deg_ref[:, 1:2] + 1.0
    dinv = lax.rsqrt(dsum)
    h1 = jnp.dot(x_ref[...], w1_ref[...], preferred_element_type=jnp.float32)
    dinv_ref[...] = dinv
    h1_ref[...] = h1
    hp1_ref[...] = h1 * dinv


def _tc2_body(s_ref, h1_ref, dinv_ref, b1_ref, g1_ref, be1_ref, w2_ref,
              h2_ref, hp2_ref):
    dinv = dinv_ref[...]
    agg = dinv * (s_ref[0] + s_ref[1]) + (dinv * dinv) * h1_ref[...] \
        + b1_ref[...]
    z = jnp.maximum(agg * g1_ref[...] + be1_ref[...], 0.0)
    h2 = jnp.dot(z, w2_ref[...], preferred_element_type=jnp.float32)
    h2_ref[...] = h2
    hp2_ref[...] = h2 * dinv


def _tc3_body(s_ref, h2_ref, dinv_ref, b2_ref, wc_ref, bc_ref, out_ref):
    dinv = dinv_ref[...]
    agg = dinv * (s_ref[0] + s_ref[1]) + (dinv * dinv) * h2_ref[...] \
        + b2_ref[...]
    out_ref[...] = jnp.dot(agg, wc_ref[...],
                           preferred_element_type=jnp.float32) + bc_ref[...]


def _row_spec(shape_minor):
    return pl.BlockSpec((_BLK,) + shape_minor, lambda i: (i,) + (0,) * len(shape_minor))


def _full_spec(shape):
    return pl.BlockSpec(shape, lambda i: (0,) * len(shape))


def _tc1(deg_t, x, w1):
    return pl.pallas_call(
        _tc1_body,
        grid=(_GRID,),
        in_specs=[_row_spec((NC,)), _row_spec((D,)), _full_spec((D, H))],
        out_specs=[_row_spec((1,)), _row_spec((H,)), _row_spec((H,))],
        out_shape=[
            jax.ShapeDtypeStruct((N, 1), jnp.float32),
            jax.ShapeDtypeStruct((N, H), jnp.float32),
            jax.ShapeDtypeStruct((N, H), jnp.float32),
        ],
    )(deg_t, x, w1)


def _tc2(s1, h1, dinv, b1, g1, be1, w2):
    sspec = pl.BlockSpec((NC, _BLK, H), lambda i: (0, i, 0))
    return pl.pallas_call(
        _tc2_body,
        grid=(_GRID,),
        in_specs=[sspec, _row_spec((H,)), _row_spec((1,)),
                  _full_spec((1, H)), _full_spec((1, H)), _full_spec((1, H)),
                  _full_spec((H, H))],
        out_specs=[_row_spec((H,)), _row_spec((H,))],
        out_shape=[
            jax.ShapeDtypeStruct((N, H), jnp.float32),
            jax.ShapeDtypeStruct((N, H), jnp.float32),
        ],
    )(s1, h1, dinv, b1, g1, be1, w2)


def _tc3(s2, h2, dinv, b2, wc, bc):
    sspec = pl.BlockSpec((NC, _BLK, H), lambda i: (0, i, 0))
    return pl.pallas_call(
        _tc3_body,
        grid=(_GRID,),
        in_specs=[sspec, _row_spec((H,)), _row_spec((1,)),
                  _full_spec((1, H)), _full_spec((H, OUT)),
                  _full_spec((1, OUT))],
        out_specs=_row_spec((OUT,)),
        out_shape=jax.ShapeDtypeStruct((N, OUT), jnp.float32),
    )(s2, h2, dinv, b2, wc, bc)


def kernel(x, edge_index, edge_weight, W1, b1, gamma1, beta1, W2, b2, Wc, bc):
    src = edge_index[0]
    dst = edge_index[1]

    dst_d = dst.reshape(NW, DCH, KD)
    ew_d = edge_weight.reshape(NW, DCH, KD)
    npad = EPTP - EPT
    pad_idx = jnp.broadcast_to((jnp.arange(npad, dtype=jnp.int32) * 41) % N,
                               (NW, npad))
    src_a = jnp.concatenate(
        [src.reshape(NW, EPT), pad_idx], axis=1).reshape(NW * EPTP)
    dst_a = jnp.concatenate(
        [dst.reshape(NW, EPT), pad_idx], axis=1).reshape(NW * EPTP)
    ew_a = jnp.concatenate(
        [edge_weight.reshape(NW, EPT),
         jnp.zeros((NW, npad), jnp.float32)], axis=1).reshape(NW * EPTP)

    deg_p = _sc_degree(dst_d, ew_d)
    deg_t = jnp.transpose(deg_p[:, :N])

    b1r = b1.reshape(1, H)
    g1r = gamma1.reshape(1, H)
    be1r = beta1.reshape(1, H)
    b2r = b2.reshape(1, H)
    bcr = bc.reshape(1, OUT)

    dinv, h1, hp1 = _tc1(deg_t, x, W1)
    s1 = _sc_aggregate(hp1, src_a, dst_a, ew_a)
    h2, hp2 = _tc2(s1, h1, dinv, b1r, g1r, be1r, W2)
    s2 = _sc_aggregate(hp2, src_a, dst_a, ew_a)
    out = _tc3(s2, h2, dinv, b2r, Wc, bcr)
    return out


# speedup vs baseline: 24.7492x; 1.2808x over previous
import functools

import jax
import jax.numpy as jnp
from jax import lax
from jax.experimental import pallas as pl
from jax.experimental.pallas import tpu as pltpu
from jax.experimental.pallas import tpu_sc as plsc

N = 10000
E = 320000
D = 128
H = 128
OUT = 70

NC = 2
NS = 16
NW = NC * NS
EPT = E // NW

KD = 100
DCH = EPT // KD
NPAD = 10240

K = 80
EPTP = 10240
NCHK = EPTP // K
NDAT = 2
NIDX = 4
NSLOT = 4

_mesh = plsc.VectorSubcoreMesh(core_axis_name="c", subcore_axis_name="s")


@functools.partial(
    pl.kernel,
    out_type=jax.ShapeDtypeStruct((NC, NPAD), jnp.float32),
    mesh=_mesh,
    scratch_types=[
        pltpu.VMEM((DCH, KD), jnp.int32),
        pltpu.VMEM((DCH, KD), jnp.float32),
        pltpu.VMEM((640,), jnp.float32),
        pltpu.VMEM_SHARED((NPAD,), jnp.float32),
        pltpu.SemaphoreType.DMA,
    ],
)
def _sc_degree(dst_hbm, ew_hbm, deg_out, dst_v, ew_v, zbuf, acc, sem):
    c = lax.axis_index("c")
    s = lax.axis_index("s")
    w = s * NC + c

    pltpu.sync_copy(dst_hbm.at[w], dst_v)
    pltpu.sync_copy(ew_hbm.at[w], ew_v)

    @pl.loop(0, 40)
    def _z(i):
        zbuf[pl.ds(i * 16, 16)] = jnp.zeros((16,), jnp.float32)

    pltpu.sync_copy(zbuf, acc.at[pl.ds(s * 640, 640)])
    plsc.subcore_barrier()

    @pl.loop(0, DCH)
    def _fire(j):
        pltpu.async_copy(ew_v.at[j], acc.at[dst_v.at[j]], sem, add=True)

    @pl.loop(0, DCH)
    def _drain(j):
        pltpu.make_async_copy(ew_v.at[0], acc.at[dst_v.at[0]], sem).wait()

    plsc.subcore_barrier()

    pltpu.sync_copy(acc.at[pl.ds(s * 640, 640)],
                    deg_out.at[c].at[pl.ds(s * 640, 640)])


@functools.partial(
    pl.kernel,
    out_type=jax.ShapeDtypeStruct((NC, N, H), jnp.float32),
    mesh=_mesh,
    scratch_types=[
        [pltpu.VMEM((K,), jnp.int32) for _ in range(NIDX)],
        [pltpu.VMEM((K,), jnp.int32) for _ in range(NIDX)],
        [pltpu.VMEM((K,), jnp.float32) for _ in range(NIDX)],
        [pltpu.VMEM((K, H), jnp.float32) for _ in range(NDAT)],
        [pltpu.VMEM((K, H), jnp.float32) for _ in range(NDAT)],
        pltpu.SemaphoreType.DMA((NIDX,)),
        pltpu.SemaphoreType.DMA((NIDX,)),
        pltpu.SemaphoreType.DMA((NDAT,)),
        pltpu.SemaphoreType.DMA((NDAT,)),
        pltpu.SemaphoreType.DMA,
        pltpu.VMEM_SHARED((N, H), jnp.float32),
    ],
)
def _sc_aggregate(hp_hbm, src_hbm, dst_hbm, ew_hbm, s_out,
                  src_b, dst_b, ew_b, gbufs, sbufs,
                  isem, dsem, gsem, ssem, zsem, acc):
    c = lax.axis_index("c")
    s = lax.axis_index("s")
    w = s * NC + c
    ebase = pl.multiple_of(w * EPTP, 8)

    def _edge_slice(ref, j):
        return ref.at[pl.ds(pl.multiple_of(ebase + j * K, 8), K)]

    for r in range(40):
        for q in range(8):
            sbufs[0][r, pl.ds(q * 16, 16)] = jnp.zeros((16,), jnp.float32)

    @pl.when(s < 10)
    def _zero():
        @pl.loop(0, 25)
        def _zf(kk):
            pltpu.async_copy(sbufs[0].at[pl.ds(0, 40)],
                             acc.at[pl.ds(s * 1000 + kk * 40, 40)], zsem)

        @pl.loop(0, 25)
        def _zd(kk):
            pltpu.make_async_copy(sbufs[0].at[pl.ds(0, 40)],
                                  acc.at[pl.ds(0, 40)], zsem).wait()

    plsc.subcore_barrier()

    for m in range(NIDX):
        pltpu.async_copy(_edge_slice(src_hbm, m), src_b[m], isem.at[m])
        pltpu.async_copy(_edge_slice(ew_hbm, m), ew_b[m], isem.at[m])
    for m in range(NDAT):
        pltpu.async_copy(_edge_slice(dst_hbm, m), dst_b[m], dsem.at[m])
    for m in range(NDAT):
        pltpu.make_async_copy(_edge_slice(src_hbm, m), src_b[m],
                              isem.at[m]).wait()
        pltpu.make_async_copy(_edge_slice(ew_hbm, m), ew_b[m],
                              isem.at[m]).wait()
        pltpu.async_copy(hp_hbm.at[src_b[m]], gbufs[m], gsem.at[m])

    @pl.loop(0, NCHK, step=NSLOT)
    def _outer(o):
        for t in range(NSLOT):
            j = o + t
            t2 = t % NDAT
            t4 = t % NIDX

            pltpu.make_async_copy(hp_hbm.at[src_b[t4]], gbufs[t2],
                                  gsem.at[t2]).wait()

            @pl.when(j >= NDAT)
            def _ws():
                pltpu.make_async_copy(sbufs[t2], acc.at[dst_b[t4]],
                                      ssem.at[t2]).wait()

            @pl.when(j + NDAT < NCHK)
            def _rd():
                m2 = (t + NDAT) % NIDX
                pltpu.async_copy(_edge_slice(dst_hbm, j + NDAT), dst_b[m2],
                                 dsem.at[m2])

            @plsc.parallel_loop(0, K // 16)
            def _grp(g):
                b16 = g * 16
                cvec = ew_b[t4][pl.ds(b16, 16)]
                for r in range(16):
                    cval = cvec[r]
                    for q in range(8):
                        sbufs[t2][b16 + r, pl.ds(q * 16, 16)] = (
                            gbufs[t2][b16 + r, pl.ds(q * 16, 16)] * cval)

            pltpu.make_async_copy(_edge_slice(dst_hbm, j), dst_b[t4],
                                  dsem.at[t4]).wait()
            pltpu.async_copy(sbufs[t2], acc.at[dst_b[t4]], ssem.at[t2],
                             add=True)

            @pl.when(j + NDAT < NCHK)
            def _g2():
                m2 = (t + NDAT) % NIDX
                pltpu.make_async_copy(_edge_slice(src_hbm, j + NDAT),
                                      src_b[m2], isem.at[m2]).wait()
                pltpu.make_async_copy(_edge_slice(ew_hbm, j + NDAT),
                                      ew_b[m2], isem.at[m2]).wait()
                pltpu.async_copy(hp_hbm.at[src_b[m2]], gbufs[t2],
                                 gsem.at[t2])

            @pl.when(j + NIDX < NCHK)
            def _ri():
                pltpu.async_copy(_edge_slice(src_hbm, j + NIDX), src_b[t4],
                                 isem.at[t4])
                pltpu.async_copy(_edge_slice(ew_hbm, j + NIDX), ew_b[t4],
                                 isem.at[t4])

    for t in range(NDAT):
        t4 = (NCHK - NDAT + t) % NIDX
        pltpu.make_async_copy(sbufs[t], acc.at[dst_b[t4]],
                              ssem.at[t]).wait()

    plsc.subcore_barrier()

    @pl.when(s < 10)
    def _wb():
        pltpu.sync_copy(acc.at[pl.ds(s * 1000, 1000)],
                        s_out.at[c].at[pl.ds(s * 1000, 1000)])


_BLK = 1000
_GRID = N // _BLK


def _tc1_body(deg_ref, x_ref, w1_ref, dinv_ref, h1_ref, hp1_ref):
    dsum = deg_ref[:, 0:1] + deg_ref[:, 1:2] + 1.0
    dinv = lax.rsqrt(dsum)
    h1 = jnp.dot(x_ref[...], w1_ref[...], preferred_element_type=jnp.float32)
    dinv_ref[...] = dinv
    h1_ref[...] = h1
    hp1_ref[...] = h1 * dinv


def _tc2_body(s_ref, h1_ref, dinv_ref, b1_ref, g1_ref, be1_ref, w2_ref,
              h2_ref, hp2_ref):
    dinv = dinv_ref[...]
    agg = dinv * (s_ref[0] + s_ref[1]) + (dinv * dinv) * h1_ref[...] \
        + b1_ref[...]
    z = jnp.maximum(agg * g1_ref[...] + be1_ref[...], 0.0)
    h2 = jnp.dot(z, w2_ref[...], preferred_element_type=jnp.float32)
    h2_ref[...] = h2
    hp2_ref[...] = h2 * dinv


def _tc3_body(s_ref, h2_ref, dinv_ref, b2_ref, wc_ref, bc_ref, out_ref):
    dinv = dinv_ref[...]
    agg = dinv * (s_ref[0] + s_ref[1]) + (dinv * dinv) * h2_ref[...] \
        + b2_ref[...]
    out_ref[...] = jnp.dot(agg, wc_ref[...],
                           preferred_element_type=jnp.float32) + bc_ref[...]


def _row_spec(shape_minor):
    return pl.BlockSpec((_BLK,) + shape_minor, lambda i: (i,) + (0,) * len(shape_minor))


def _full_spec(shape):
    return pl.BlockSpec(shape, lambda i: (0,) * len(shape))


def _tc1(deg_t, x, w1):
    return pl.pallas_call(
        _tc1_body,
        grid=(_GRID,),
        in_specs=[_row_spec((NC,)), _row_spec((D,)), _full_spec((D, H))],
        out_specs=[_row_spec((1,)), _row_spec((H,)), _row_spec((H,))],
        out_shape=[
            jax.ShapeDtypeStruct((N, 1), jnp.float32),
            jax.ShapeDtypeStruct((N, H), jnp.float32),
            jax.ShapeDtypeStruct((N, H), jnp.float32),
        ],
    )(deg_t, x, w1)


def _tc2(s1, h1, dinv, b1, g1, be1, w2):
    sspec = pl.BlockSpec((NC, _BLK, H), lambda i: (0, i, 0))
    return pl.pallas_call(
        _tc2_body,
        grid=(_GRID,),
        in_specs=[sspec, _row_spec((H,)), _row_spec((1,)),
                  _full_spec((1, H)), _full_spec((1, H)), _full_spec((1, H)),
                  _full_spec((H, H))],
        out_specs=[_row_spec((H,)), _row_spec((H,))],
        out_shape=[
            jax.ShapeDtypeStruct((N, H), jnp.float32),
            jax.ShapeDtypeStruct((N, H), jnp.float32),
        ],
    )(s1, h1, dinv, b1, g1, be1, w2)


def _tc3(s2, h2, dinv, b2, wc, bc):
    sspec = pl.BlockSpec((NC, _BLK, H), lambda i: (0, i, 0))
    return pl.pallas_call(
        _tc3_body,
        grid=(_GRID,),
        in_specs=[sspec, _row_spec((H,)), _row_spec((1,)),
                  _full_spec((1, H)), _full_spec((H, OUT)),
                  _full_spec((1, OUT))],
        out_specs=_row_spec((OUT,)),
        out_shape=jax.ShapeDtypeStruct((N, OUT), jnp.float32),
    )(s2, h2, dinv, b2, wc, bc)


def kernel(x, edge_index, edge_weight, W1, b1, gamma1, beta1, W2, b2, Wc, bc):
    src = edge_index[0]
    dst = edge_index[1]

    dst_d = dst.reshape(NW, DCH, KD)
    ew_d = edge_weight.reshape(NW, DCH, KD)
    npad = EPTP - EPT
    pad_idx = jnp.broadcast_to((jnp.arange(npad, dtype=jnp.int32) * 41) % N,
                               (NW, npad))
    src_a = jnp.concatenate(
        [src.reshape(NW, EPT), pad_idx], axis=1).reshape(NW * EPTP)
    dst_a = jnp.concatenate(
        [dst.reshape(NW, EPT), pad_idx], axis=1).reshape(NW * EPTP)
    ew_a = jnp.concatenate(
        [edge_weight.reshape(NW, EPT),
         jnp.zeros((NW, npad), jnp.float32)], axis=1).reshape(NW * EPTP)

    deg_p = _sc_degree(dst_d, ew_d)
    deg_t = jnp.transpose(deg_p[:, :N])

    b1r = b1.reshape(1, H)
    g1r = gamma1.reshape(1, H)
    be1r = beta1.reshape(1, H)
    b2r = b2.reshape(1, H)
    bcr = bc.reshape(1, OUT)

    dinv, h1, hp1 = _tc1(deg_t, x, W1)
    s1 = _sc_aggregate(hp1, src_a, dst_a, ew_a)
    h2, hp2 = _tc2(s1, h1, dinv, b1r, g1r, be1r, W2)
    s2 = _sc_aggregate(hp2, src_a, dst_a, ew_a)
    out = _tc3(s2, h2, dinv, b2r, Wc, bcr)
    return out


# speedup vs baseline: 27.1898x; 1.0986x over previous
import functools

import jax
import jax.numpy as jnp
from jax import lax
from jax.experimental import pallas as pl
from jax.experimental.pallas import tpu as pltpu
from jax.experimental.pallas import tpu_sc as plsc

N = 10000
E = 320000
D = 128
H = 128
OUT = 70

NC = 2
NS = 16
NW = NC * NS
EPT = E // NW

KD = 100
DCH = EPT // KD
NPAD = 10240

K = 80
EPTP = 10240
NCHK = EPTP // K
NDAT = 2
NIDX = 4
NSLOT = 4

_mesh = plsc.VectorSubcoreMesh(core_axis_name="c", subcore_axis_name="s")


@functools.partial(
    pl.kernel,
    out_type=jax.ShapeDtypeStruct((NC, NPAD), jnp.float32),
    mesh=_mesh,
    scratch_types=[
        pltpu.VMEM((DCH, KD), jnp.int32),
        pltpu.VMEM((DCH, KD), jnp.float32),
        pltpu.VMEM((640,), jnp.float32),
        pltpu.VMEM_SHARED((NPAD,), jnp.float32),
        pltpu.SemaphoreType.DMA,
    ],
)
def _sc_degree(dst_hbm, ew_hbm, deg_out, dst_v, ew_v, zbuf, acc, sem):
    c = lax.axis_index("c")
    s = lax.axis_index("s")
    w = s * NC + c

    pltpu.sync_copy(dst_hbm.at[w], dst_v)
    pltpu.sync_copy(ew_hbm.at[w], ew_v)

    @pl.loop(0, 40)
    def _z(i):
        zbuf[pl.ds(i * 16, 16)] = jnp.zeros((16,), jnp.float32)

    pltpu.sync_copy(zbuf, acc.at[pl.ds(s * 640, 640)])
    plsc.subcore_barrier()

    @pl.loop(0, DCH)
    def _fire(j):
        pltpu.async_copy(ew_v.at[j], acc.at[dst_v.at[j]], sem, add=True)

    @pl.loop(0, DCH)
    def _drain(j):
        pltpu.make_async_copy(ew_v.at[0], acc.at[dst_v.at[0]], sem).wait()

    plsc.subcore_barrier()

    pltpu.sync_copy(acc.at[pl.ds(s * 640, 640)],
                    deg_out.at[c].at[pl.ds(s * 640, 640)])


@functools.partial(
    pl.kernel,
    out_type=jax.ShapeDtypeStruct((NC, N, H), jnp.float32),
    mesh=_mesh,
    scratch_types=[
        [pltpu.VMEM((K,), jnp.int32) for _ in range(NIDX)],
        [pltpu.VMEM((K,), jnp.int32) for _ in range(NIDX)],
        [pltpu.VMEM((K,), jnp.float32) for _ in range(NIDX)],
        [pltpu.VMEM((K, H), jnp.float32) for _ in range(NIDX)],
        pltpu.SemaphoreType.DMA((NIDX,)),
        pltpu.SemaphoreType.DMA((NIDX,)),
        pltpu.SemaphoreType.DMA((NIDX,)),
        pltpu.SemaphoreType.DMA((NIDX,)),
        pltpu.SemaphoreType.DMA,
        pltpu.VMEM_SHARED((N, H), jnp.float32),
    ],
)
def _sc_aggregate(hp_hbm, src_hbm, dst_hbm, ew_hbm, s_out,
                  src_b, dst_b, ew_b, gbufs,
                  isem, dsem, gsem, ssem, zsem, acc):
    c = lax.axis_index("c")
    s = lax.axis_index("s")
    w = s * NC + c
    ebase = pl.multiple_of(w * EPTP, 8)

    def _edge_slice(ref, j):
        return ref.at[pl.ds(pl.multiple_of(ebase + j * K, 8), K)]

    for r in range(40):
        for q in range(8):
            gbufs[0][r, pl.ds(q * 16, 16)] = jnp.zeros((16,), jnp.float32)

    @pl.when(s < 10)
    def _zero():
        @pl.loop(0, 25)
        def _zf(kk):
            pltpu.async_copy(gbufs[0].at[pl.ds(0, 40)],
                             acc.at[pl.ds(s * 1000 + kk * 40, 40)], zsem)

        @pl.loop(0, 25)
        def _zd(kk):
            pltpu.make_async_copy(gbufs[0].at[pl.ds(0, 40)],
                                  acc.at[pl.ds(0, 40)], zsem).wait()

    plsc.subcore_barrier()

    for m in range(NIDX):
        pltpu.async_copy(_edge_slice(src_hbm, m), src_b[m], isem.at[m])
        pltpu.async_copy(_edge_slice(ew_hbm, m), ew_b[m], isem.at[m])
    for m in range(3):
        pltpu.async_copy(_edge_slice(dst_hbm, m), dst_b[m], dsem.at[m])
    for m in range(3):
        pltpu.make_async_copy(_edge_slice(src_hbm, m), src_b[m],
                              isem.at[m]).wait()
        pltpu.make_async_copy(_edge_slice(ew_hbm, m), ew_b[m],
                              isem.at[m]).wait()
        pltpu.async_copy(hp_hbm.at[src_b[m]], gbufs[m], gsem.at[m])

    @pl.loop(0, NCHK, step=NSLOT)
    def _outer(o):
        for t in range(NSLOT):
            j = o + t
            t3 = (t + 3) % NIDX

            pltpu.make_async_copy(hp_hbm.at[src_b[t]], gbufs[t],
                                  gsem.at[t]).wait()

            @plsc.parallel_loop(0, K // 16)
            def _grp(g):
                b16 = g * 16
                cvec = ew_b[t][pl.ds(b16, 16)]
                for r in range(16):
                    cval = cvec[r]
                    for q in range(8):
                        gbufs[t][b16 + r, pl.ds(q * 16, 16)] = (
                            gbufs[t][b16 + r, pl.ds(q * 16, 16)] * cval)

            pltpu.make_async_copy(_edge_slice(dst_hbm, j), dst_b[t],
                                  dsem.at[t]).wait()
            pltpu.async_copy(gbufs[t], acc.at[dst_b[t]], ssem.at[t],
                             add=True)

            @pl.when((j >= 1) & (j + 3 < NCHK))
            def _ws():
                pltpu.make_async_copy(gbufs[t3], acc.at[dst_b[t3]],
                                      ssem.at[t3]).wait()

            @pl.when(j + 3 < NCHK)
            def _g3():
                pltpu.async_copy(_edge_slice(dst_hbm, j + 3), dst_b[t3],
                                 dsem.at[t3])
                pltpu.make_async_copy(_edge_slice(src_hbm, j + 3),
                                      src_b[t3], isem.at[t3]).wait()
                pltpu.make_async_copy(_edge_slice(ew_hbm, j + 3),
                                      ew_b[t3], isem.at[t3]).wait()
                pltpu.async_copy(hp_hbm.at[src_b[t3]], gbufs[t3],
                                 gsem.at[t3])

            @pl.when(j + NIDX < NCHK)
            def _ri():
                pltpu.async_copy(_edge_slice(src_hbm, j + NIDX), src_b[t],
                                 isem.at[t])
                pltpu.async_copy(_edge_slice(ew_hbm, j + NIDX), ew_b[t],
                                 isem.at[t])

    for t in range(NIDX):
        pltpu.make_async_copy(gbufs[t], acc.at[dst_b[t]],
                              ssem.at[t]).wait()

    plsc.subcore_barrier()

    @pl.when(s < 10)
    def _wb():
        pltpu.sync_copy(acc.at[pl.ds(s * 1000, 1000)],
                        s_out.at[c].at[pl.ds(s * 1000, 1000)])


_BLK = 1000
_GRID = N // _BLK


def _tc1_body(deg_ref, x_ref, w1_ref, dinv_ref, h1_ref, hp1_ref):
    dsum = deg_ref[:, 0:1] + deg_ref[:, 1:2] + 1.0
    dinv = lax.rsqrt(dsum)
    h1 = jnp.dot(x_ref[...], w1_ref[...], preferred_element_type=jnp.float32)
    dinv_ref[...] = dinv
    h1_ref[...] = h1
    hp1_ref[...] = h1 * dinv


def _tc2_body(s_ref, h1_ref, dinv_ref, b1_ref, g1_ref, be1_ref, w2_ref,
              h2_ref, hp2_ref):
    dinv = dinv_ref[...]
    agg = dinv * (s_ref[0] + s_ref[1]) + (dinv * dinv) * h1_ref[...] \
        + b1_ref[...]
    z = jnp.maximum(agg * g1_ref[...] + be1_ref[...], 0.0)
    h2 = jnp.dot(z, w2_ref[...], preferred_element_type=jnp.float32)
    h2_ref[...] = h2
    hp2_ref[...] = h2 * dinv


def _tc3_body(s_ref, h2_ref, dinv_ref, b2_ref, wc_ref, bc_ref, out_ref):
    dinv = dinv_ref[...]
    agg = dinv * (s_ref[0] + s_ref[1]) + (dinv * dinv) * h2_ref[...] \
        + b2_ref[...]
    out_ref[...] = jnp.dot(agg, wc_ref[...],
                           preferred_element_type=jnp.float32) + bc_ref[...]


def _row_spec(shape_minor):
    return pl.BlockSpec((_BLK,) + shape_minor, lambda i: (i,) + (0,) * len(shape_minor))


def _full_spec(shape):
    return pl.BlockSpec(shape, lambda i: (0,) * len(shape))


def _tc1(deg_t, x, w1):
    return pl.pallas_call(
        _tc1_body,
        grid=(_GRID,),
        in_specs=[_row_spec((NC,)), _row_spec((D,)), _full_spec((D, H))],
        out_specs=[_row_spec((1,)), _row_spec((H,)), _row_spec((H,))],
        out_shape=[
            jax.ShapeDtypeStruct((N, 1), jnp.float32),
            jax.ShapeDtypeStruct((N, H), jnp.float32),
            jax.ShapeDtypeStruct((N, H), jnp.float32),
        ],
    )(deg_t, x, w1)


def _tc2(s1, h1, dinv, b1, g1, be1, w2):
    sspec = pl.BlockSpec((NC, _BLK, H), lambda i: (0, i, 0))
    return pl.pallas_call(
        _tc2_body,
        grid=(_GRID,),
        in_specs=[sspec, _row_spec((H,)), _row_spec((1,)),
                  _full_spec((1, H)), _full_spec((1, H)), _full_spec((1, H)),
                  _full_spec((H, H))],
        out_specs=[_row_spec((H,)), _row_spec((H,))],
        out_shape=[
            jax.ShapeDtypeStruct((N, H), jnp.float32),
            jax.ShapeDtypeStruct((N, H), jnp.float32),
        ],
    )(s1, h1, dinv, b1, g1, be1, w2)


def _tc3(s2, h2, dinv, b2, wc, bc):
    sspec = pl.BlockSpec((NC, _BLK, H), lambda i: (0, i, 0))
    return pl.pallas_call(
        _tc3_body,
        grid=(_GRID,),
        in_specs=[sspec, _row_spec((H,)), _row_spec((1,)),
                  _full_spec((1, H)), _full_spec((H, OUT)),
                  _full_spec((1, OUT))],
        out_specs=_row_spec((OUT,)),
        out_shape=jax.ShapeDtypeStruct((N, OUT), jnp.float32),
    )(s2, h2, dinv, b2, wc, bc)


def kernel(x, edge_index, edge_weight, W1, b1, gamma1, beta1, W2, b2, Wc, bc):
    src = edge_index[0]
    dst = edge_index[1]

    dst_d = dst.reshape(NW, DCH, KD)
    ew_d = edge_weight.reshape(NW, DCH, KD)
    npad = EPTP - EPT
    pad_idx = jnp.broadcast_to((jnp.arange(npad, dtype=jnp.int32) * 41) % N,
                               (NW, npad))
    src_a = jnp.concatenate(
        [src.reshape(NW, EPT), pad_idx], axis=1).reshape(NW * EPTP)
    dst_a = jnp.concatenate(
        [dst.reshape(NW, EPT), pad_idx], axis=1).reshape(NW * EPTP)
    ew_a = jnp.concatenate(
        [edge_weight.reshape(NW, EPT),
         jnp.zeros((NW, npad), jnp.float32)], axis=1).reshape(NW * EPTP)

    deg_p = _sc_degree(dst_d, ew_d)
    deg_t = jnp.transpose(deg_p[:, :N])

    b1r = b1.reshape(1, H)
    g1r = gamma1.reshape(1, H)
    be1r = beta1.reshape(1, H)
    b2r = b2.reshape(1, H)
    bcr = bc.reshape(1, OUT)

    dinv, h1, hp1 = _tc1(deg_t, x, W1)
    s1 = _sc_aggregate(hp1, src_a, dst_a, ew_a)
    h2, hp2 = _tc2(s1, h1, dinv, b1r, g1r, be1r, W2)
    s2 = _sc_aggregate(hp2, src_a, dst_a, ew_a)
    out = _tc3(s2, h2, dinv, b2r, Wc, bcr)
    return out
